# baseline (speedup 1.0000x reference)
import functools

import jax
import jax.numpy as jnp
from jax import lax
from jax.experimental import pallas as pl
from jax.experimental.pallas import tpu as pltpu
from jax.experimental.pallas import tpu_sc as plsc

GAMMA = 10.0
N, K, NF = 10000, 16, 300
LOG2E = 1.4426950408889634
SQG = 3.798282560433022
C0 = 0.5287663729448977

NC, NS = 2, 16
NW = NC * NS
BN = 400
NBLK = N // BN
EB = BN * K


def _sc_dist(xyz, idx):
    mesh = plsc.VectorSubcoreMesh(core_axis_name="c", subcore_axis_name="s")

    @functools.partial(
        pl.kernel,
        mesh=mesh,
        out_type=jax.ShapeDtypeStruct((N * K,), jnp.float32),
        scratch_types=[
            pltpu.VMEM((N, 3), jnp.float32),
            pltpu.VMEM((EB,), jnp.int32),
            pltpu.VMEM((EB,), jnp.float32),
        ],
        compiler_params=pltpu.CompilerParams(use_tc_tiling_on_sc=False,
                                             needs_layout_passes=False),
    )
    def k(x_hbm, idx_hbm, out_hbm, xyzv, idx_v, d_v):
        wid = lax.axis_index("s") * NC + lax.axis_index("c")

        @pl.when(wid < NBLK)
        def _():
            pltpu.sync_copy(x_hbm, xyzv)
            pltpu.sync_copy(idx_hbm.at[pl.ds(wid * EB, EB)], idx_v)
            node0 = wid * BN
            lanes = lax.iota(jnp.int32, K)
            c0 = jnp.broadcast_to(jnp.int32(0), (K,))
            c1 = jnp.broadcast_to(jnp.int32(1), (K,))
            c2 = jnp.broadcast_to(jnp.int32(2), (K,))

            def body(g, carry):
                nbr = idx_v[pl.ds(g * K, K)]
                n = jnp.broadcast_to(node0 + g, (K,)).astype(jnp.int32)
                dx = (plsc.load_gather(xyzv, [nbr, c0])
                      - plsc.load_gather(xyzv, [n, c0]))
                dy = (plsc.load_gather(xyzv, [nbr, c1])
                      - plsc.load_gather(xyzv, [n, c1]))
                dz = (plsc.load_gather(xyzv, [nbr, c2])
                      - plsc.load_gather(xyzv, [n, c2]))
                s2 = dx * dx + dy * dy + dz * dz + 1e-12
                y = plsc.bitcast(
                    jnp.int32(0x5F3759DF)
                    - lax.shift_right_arithmetic(plsc.bitcast(s2, jnp.int32),
                                                 1), jnp.float32)
                hx = 0.5 * s2
                for _ in range(4):
                    y = y * (1.5 - hx * y * y)
                pos = lanes * BN + g
                plsc.store_scatter(d_v, [pos], (SQG * s2) * y)
                return carry

            lax.fori_loop(0, BN, body, 0)
            pltpu.sync_copy(d_v, out_hbm.at[pl.ds(wid * EB, EB)])

    return k(xyz, idx)


def _tc_body(x_ref, d_ref, cen_ref,
             wp_ref, bp_ref, w1_ref, b1_ref, w2_ref, b2_ref,
             w3_ref, b3_ref, w4_ref, b4_ref, out_ref, *, bn):
    def g_act(v):
        return jnp.log2(jnp.exp2(v) + 0.5)

    x = x_ref[...]
    u = d_ref[...]
    b1 = b1_ref[...] * LOG2E - 1.0
    b2 = b2_ref[...] * LOG2E - 1.0
    b3 = b3_ref[...] * LOG2E - 1.0
    w = u - cen_ref[...]
    rbf = jnp.exp2(C0 - w * w)
    g1 = g_act(jnp.dot(rbf, w1_ref[...],
                       preferred_element_type=jnp.float32) + b1)
    g2 = g_act(jnp.dot(g1, w2_ref[...],
                       preferred_element_type=jnp.float32) + b2)
    s = jnp.sum(g2.reshape(K, bn, NF), axis=0)
    pre = jnp.dot(x, wp_ref[...],
                  preferred_element_type=jnp.float32) + bp_ref[...]
    msg = pre * s
    g3 = jnp.log(jnp.exp2(jnp.dot(msg, w3_ref[...],
                                  preferred_element_type=jnp.float32) + b3)
                 + 0.5)
    post = jnp.dot(g3, w4_ref[...], preferred_element_type=jnp.float32)
    out_ref[...] = x + post + b4_ref[...]


def _tc_main(x, d, cen, wp, bp, w1, b1, w2, b2, w3, b3, w4, b4):
    full = lambda i: (0, 0)
    return pl.pallas_call(
        functools.partial(_tc_body, bn=BN),
        grid=(NBLK,),
        in_specs=[
            pl.BlockSpec((BN, NF), lambda i: (i, 0)),
            pl.BlockSpec((EB, 1), lambda i: (i, 0)),
            pl.BlockSpec((1, NF), full),
            pl.BlockSpec((NF, NF), full),
            pl.BlockSpec((1, NF), full),
            pl.BlockSpec((NF, NF), full),
            pl.BlockSpec((1, NF), full),
            pl.BlockSpec((NF, NF), full),
            pl.BlockSpec((1, NF), full),
            pl.BlockSpec((NF, NF), full),
            pl.BlockSpec((1, NF), full),
            pl.BlockSpec((NF, NF), full),
            pl.BlockSpec((1, NF), full),
        ],
        out_specs=pl.BlockSpec((BN, NF), lambda i: (i, 0)),
        out_shape=jax.ShapeDtypeStruct((N, NF), jnp.float32),
        compiler_params=pltpu.CompilerParams(
            dimension_semantics=("arbitrary",)),
    )(x, d, cen, wp, bp, w1, b1, w2, b2, w3, b3, w4, b4)


def kernel(x, xyz, nbr_idx, W_pre, b_pre, W1, b1, W2, b2, W3, b3, W4, b4):
    idx = nbr_idx.astype(jnp.int32).reshape(-1)
    d = _sc_dist(xyz.astype(jnp.float32), idx).reshape(N * K, 1)
    cen = (jnp.linspace(0.1, 30.1, NF).astype(jnp.float32)
           * SQG).reshape(1, NF)
    return _tc_main(x, d, cen,
                    W_pre, b_pre.reshape(1, NF),
                    W1, b1.reshape(1, NF), W2, b2.reshape(1, NF),
                    W3, b3.reshape(1, NF), W4, b4.reshape(1, NF))


# speedup vs baseline: 1.0367x; 1.0367x over previous
import functools

import jax
import jax.numpy as jnp
from jax import lax
from jax.experimental import pallas as pl
from jax.experimental.pallas import tpu as pltpu
from jax.experimental.pallas import tpu_sc as plsc

GAMMA = 10.0
N, K, NF = 10000, 16, 300
LOG2E = 1.4426950408889634
SQG = 3.798282560433022
C0 = 0.5287663729448977
EPS2 = 1e-12 * SQG * SQG

NC, NS = 2, 16
NW = NC * NS
BN = 400
NBLK = N // BN
EB = BN * K


def _sc_dist(xyz, idx):
    mesh = plsc.VectorSubcoreMesh(core_axis_name="c", subcore_axis_name="s")

    @functools.partial(
        pl.kernel,
        mesh=mesh,
        out_type=jax.ShapeDtypeStruct((N * K,), jnp.float32),
        scratch_types=[
            pltpu.VMEM((N,), jnp.float32),
            pltpu.VMEM((N,), jnp.float32),
            pltpu.VMEM((N,), jnp.float32),
            pltpu.VMEM((EB,), jnp.int32),
            pltpu.VMEM((EB,), jnp.float32),
        ],
        compiler_params=pltpu.CompilerParams(use_tc_tiling_on_sc=False,
                                             needs_layout_passes=False),
    )
    def k(x_hbm, idx_hbm, out_hbm, xv, yv, zv, idx_v, d_v):
        wid = lax.axis_index("s") * NC + lax.axis_index("c")

        @pl.when(wid < NBLK)
        def _():
            pltpu.sync_copy(x_hbm.at[0], xv)
            pltpu.sync_copy(x_hbm.at[1], yv)
            pltpu.sync_copy(x_hbm.at[2], zv)
            pltpu.sync_copy(idx_hbm.at[pl.ds(wid * EB, EB)], idx_v)
            node0 = wid * BN
            lanes = lax.iota(jnp.int32, K)

            def body(g, carry):
                nbr = idx_v[pl.ds(g * K, K)]
                n = jnp.broadcast_to(node0 + g, (K,)).astype(jnp.int32)
                dx = plsc.load_gather(xv, [nbr]) - plsc.load_gather(xv, [n])
                dy = plsc.load_gather(yv, [nbr]) - plsc.load_gather(yv, [n])
                dz = plsc.load_gather(zv, [nbr]) - plsc.load_gather(zv, [n])
                s2 = dx * dx + dy * dy + dz * dz + EPS2
                y = plsc.bitcast(
                    jnp.int32(0x5F3759DF)
                    - lax.shift_right_arithmetic(plsc.bitcast(s2, jnp.int32),
                                                 1), jnp.float32)
                hx = 0.5 * s2
                for _ in range(4):
                    y = y * (1.5 - hx * y * y)
                pos = lanes * BN + g
                plsc.store_scatter(d_v, [pos], s2 * y)
                return carry

            lax.fori_loop(0, BN, body, 0)
            pltpu.sync_copy(d_v, out_hbm.at[pl.ds(wid * EB, EB)])

    return k(xyz, idx)


def _tc_body(x_ref, d_ref, cen_ref,
             wp_ref, bp_ref, w1_ref, b1_ref, w2_ref, b2_ref,
             w3_ref, b3_ref, w4_ref, b4_ref, out_ref, *, bn):
    def g_act(v):
        return jnp.log2(jnp.exp2(v) + 0.5)

    x = x_ref[...]
    u = d_ref[...]
    b1 = b1_ref[...] * LOG2E - 1.0
    b2 = b2_ref[...] * LOG2E - 1.0
    b3 = b3_ref[...] * LOG2E - 1.0
    w = u - cen_ref[...]
    rbf = jnp.exp2(C0 - w * w)
    g1 = g_act(jnp.dot(rbf, w1_ref[...],
                       preferred_element_type=jnp.float32) + b1)
    g2 = g_act(jnp.dot(g1, w2_ref[...],
                       preferred_element_type=jnp.float32) + b2)
    s = jnp.sum(g2.reshape(K, bn, NF), axis=0)
    pre = jnp.dot(x, wp_ref[...],
                  preferred_element_type=jnp.float32) + bp_ref[...]
    msg = pre * s
    g3 = jnp.log(jnp.exp2(jnp.dot(msg, w3_ref[...],
                                  preferred_element_type=jnp.float32) + b3)
                 + 0.5)
    post = jnp.dot(g3, w4_ref[...], preferred_element_type=jnp.float32)
    out_ref[...] = x + post + b4_ref[...]


def _tc_main(x, d, cen, wp, bp, w1, b1, w2, b2, w3, b3, w4, b4):
    full = lambda i: (0, 0)
    return pl.pallas_call(
        functools.partial(_tc_body, bn=BN),
        grid=(NBLK,),
        in_specs=[
            pl.BlockSpec((BN, NF), lambda i: (i, 0)),
            pl.BlockSpec((EB, 1), lambda i: (i, 0)),
            pl.BlockSpec((1, NF), full),
            pl.BlockSpec((NF, NF), full),
            pl.BlockSpec((1, NF), full),
            pl.BlockSpec((NF, NF), full),
            pl.BlockSpec((1, NF), full),
            pl.BlockSpec((NF, NF), full),
            pl.BlockSpec((1, NF), full),
            pl.BlockSpec((NF, NF), full),
            pl.BlockSpec((1, NF), full),
            pl.BlockSpec((NF, NF), full),
            pl.BlockSpec((1, NF), full),
        ],
        out_specs=pl.BlockSpec((BN, NF), lambda i: (i, 0)),
        out_shape=jax.ShapeDtypeStruct((N, NF), jnp.float32),
        compiler_params=pltpu.CompilerParams(
            dimension_semantics=("arbitrary",)),
    )(x, d, cen, wp, bp, w1, b1, w2, b2, w3, b3, w4, b4)


def kernel(x, xyz, nbr_idx, W_pre, b_pre, W1, b1, W2, b2, W3, b3, W4, b4):
    idx = nbr_idx.astype(jnp.int32).reshape(-1)
    xcols = xyz.astype(jnp.float32).T * SQG
    d = _sc_dist(xcols, idx).reshape(N * K, 1)
    cen = (jnp.linspace(0.1, 30.1, NF).astype(jnp.float32)
           * SQG).reshape(1, NF)
    return _tc_main(x, d, cen,
                    W_pre, b_pre.reshape(1, NF),
                    W1, b1.reshape(1, NF), W2, b2.reshape(1, NF),
                    W3, b3.reshape(1, NF), W4, b4.reshape(1, NF))


# speedup vs baseline: 1.1784x; 1.1367x over previous
import functools

import jax
import jax.numpy as jnp
from jax import lax
from jax.experimental import pallas as pl
from jax.experimental.pallas import tpu as pltpu
from jax.experimental.pallas import tpu_sc as plsc

GAMMA = 10.0
N, K, NF = 10000, 16, 300
LOG2E = 1.4426950408889634
SQG = 3.798282560433022
C0 = 0.5287663729448977
EPS2 = 1e-12 * SQG * SQG

NC, NS = 2, 16
NW = NC * NS
BN = 400
NBLK = N // BN
EB = BN * K


def _sc_dist(xyz, idx):
    mesh = plsc.VectorSubcoreMesh(core_axis_name="c", subcore_axis_name="s")

    @functools.partial(
        pl.kernel,
        mesh=mesh,
        out_type=jax.ShapeDtypeStruct((NBLK, 1, EB), jnp.float32),
        scratch_types=[
            pltpu.VMEM((N,), jnp.float32),
            pltpu.VMEM((N,), jnp.float32),
            pltpu.VMEM((N,), jnp.float32),
            pltpu.VMEM((EB,), jnp.int32),
            pltpu.VMEM((EB,), jnp.float32),
        ],
        compiler_params=pltpu.CompilerParams(use_tc_tiling_on_sc=False,
                                             needs_layout_passes=False),
    )
    def k(x_hbm, idx_hbm, out_hbm, xv, yv, zv, idx_v, d_v):
        wid = lax.axis_index("s") * NC + lax.axis_index("c")

        @pl.when(wid < NBLK)
        def _():
            pltpu.sync_copy(x_hbm.at[0], xv)
            pltpu.sync_copy(x_hbm.at[1], yv)
            pltpu.sync_copy(x_hbm.at[2], zv)
            pltpu.sync_copy(idx_hbm.at[pl.ds(wid * EB, EB)], idx_v)
            node0 = wid * BN
            lanes = lax.iota(jnp.int32, K)

            def body(g, carry):
                nbr = idx_v[pl.ds(g * K, K)]
                n = jnp.broadcast_to(node0 + g, (K,)).astype(jnp.int32)
                dx = plsc.load_gather(xv, [nbr]) - plsc.load_gather(xv, [n])
                dy = plsc.load_gather(yv, [nbr]) - plsc.load_gather(yv, [n])
                dz = plsc.load_gather(zv, [nbr]) - plsc.load_gather(zv, [n])
                s2 = dx * dx + dy * dy + dz * dz + EPS2
                y = plsc.bitcast(
                    jnp.int32(0x5F3759DF)
                    - lax.shift_right_arithmetic(plsc.bitcast(s2, jnp.int32),
                                                 1), jnp.float32)
                hx = 0.5 * s2
                for _ in range(4):
                    y = y * (1.5 - hx * y * y)
                pos = lanes * BN + g
                plsc.store_scatter(d_v, [pos], s2 * y)
                return carry

            lax.fori_loop(0, BN, body, 0)
            pltpu.sync_copy(d_v, out_hbm.at[wid, 0])

    return k(xyz, idx)


def _tc_body(x_ref, d_ref, cen_ref,
             wp_ref, bp_ref, w1_ref, b1_ref, w2_ref, b2_ref,
             w3_ref, b3_ref, w4_ref, b4_ref, out_ref, *, bn):
    def g_act(v):
        return jnp.log2(jnp.exp2(v) + 0.5)

    x = x_ref[...]
    u = jnp.transpose(d_ref[0])
    b1 = b1_ref[...] * LOG2E - 1.0
    b2 = b2_ref[...] * LOG2E - 1.0
    b3 = b3_ref[...] * LOG2E - 1.0
    w = u - cen_ref[...]
    rbf = jnp.exp2(C0 - w * w)
    g1 = g_act(jnp.dot(rbf, w1_ref[...],
                       preferred_element_type=jnp.float32) + b1)
    g2 = g_act(jnp.dot(g1, w2_ref[...],
                       preferred_element_type=jnp.float32) + b2)
    s = jnp.sum(g2.reshape(K, bn, NF), axis=0)
    pre = jnp.dot(x, wp_ref[...],
                  preferred_element_type=jnp.float32) + bp_ref[...]
    msg = pre * s
    g3 = jnp.log(jnp.exp2(jnp.dot(msg, w3_ref[...],
                                  preferred_element_type=jnp.float32) + b3)
                 + 0.5)
    post = jnp.dot(g3, w4_ref[...], preferred_element_type=jnp.float32)
    out_ref[...] = x + post + b4_ref[...]


def _tc_main(x, d, cen, wp, bp, w1, b1, w2, b2, w3, b3, w4, b4):
    full = lambda i: (0, 0)
    return pl.pallas_call(
        functools.partial(_tc_body, bn=BN),
        grid=(NBLK,),
        in_specs=[
            pl.BlockSpec((BN, NF), lambda i: (i, 0)),
            pl.BlockSpec((1, 1, EB), lambda i: (i, 0, 0)),
            pl.BlockSpec((1, NF), full),
            pl.BlockSpec((NF, NF), full),
            pl.BlockSpec((1, NF), full),
            pl.BlockSpec((NF, NF), full),
            pl.BlockSpec((1, NF), full),
            pl.BlockSpec((NF, NF), full),
            pl.BlockSpec((1, NF), full),
            pl.BlockSpec((NF, NF), full),
            pl.BlockSpec((1, NF), full),
            pl.BlockSpec((NF, NF), full),
            pl.BlockSpec((1, NF), full),
        ],
        out_specs=pl.BlockSpec((BN, NF), lambda i: (i, 0)),
        out_shape=jax.ShapeDtypeStruct((N, NF), jnp.float32),
        compiler_params=pltpu.CompilerParams(
            dimension_semantics=("arbitrary",)),
    )(x, d, cen, wp, bp, w1, b1, w2, b2, w3, b3, w4, b4)


def kernel(x, xyz, nbr_idx, W_pre, b_pre, W1, b1, W2, b2, W3, b3, W4, b4):
    idx = nbr_idx.astype(jnp.int32).reshape(-1)
    xcols = xyz.astype(jnp.float32).T * SQG
    d = _sc_dist(xcols, idx)
    cen = (jnp.linspace(0.1, 30.1, NF).astype(jnp.float32)
           * SQG).reshape(1, NF)
    return _tc_main(x, d, cen,
                    W_pre, b_pre.reshape(1, NF),
                    W1, b1.reshape(1, NF), W2, b2.reshape(1, NF),
                    W3, b3.reshape(1, NF), W4, b4.reshape(1, NF))


# speedup vs baseline: 1.2104x; 1.0271x over previous
import functools

import jax
import jax.numpy as jnp
from jax import lax
from jax.experimental import pallas as pl
from jax.experimental.pallas import tpu as pltpu
from jax.experimental.pallas import tpu_sc as plsc

GAMMA = 10.0
N, K, NF = 10000, 16, 300
LOG2E = 1.4426950408889634
SQG = 3.798282560433022
C0 = 0.5287663729448977
EPS2 = 1e-12 * SQG * SQG

NC, NS = 2, 16
NW = NC * NS
BN = 400
NBLK = N // BN
EB = BN * K


def _sc_dist(xyz, idx):
    mesh = plsc.VectorSubcoreMesh(core_axis_name="c", subcore_axis_name="s")

    @functools.partial(
        pl.kernel,
        mesh=mesh,
        out_type=jax.ShapeDtypeStruct((NBLK, 1, EB), jnp.float32),
        scratch_types=[
            pltpu.VMEM((N,), jnp.float32),
            pltpu.VMEM((N,), jnp.float32),
            pltpu.VMEM((N,), jnp.float32),
            pltpu.VMEM((EB,), jnp.int32),
            pltpu.VMEM((EB,), jnp.float32),
        ],
        compiler_params=pltpu.CompilerParams(use_tc_tiling_on_sc=False,
                                             needs_layout_passes=False),
    )
    def k(x_hbm, idx_hbm, out_hbm, xv, yv, zv, idx_v, d_v):
        wid = lax.axis_index("s") * NC + lax.axis_index("c")

        @pl.when(wid < NBLK)
        def _():
            pltpu.sync_copy(x_hbm.at[0], xv)
            pltpu.sync_copy(x_hbm.at[1], yv)
            pltpu.sync_copy(x_hbm.at[2], zv)
            pltpu.sync_copy(idx_hbm.at[pl.ds(wid * EB, EB)], idx_v)
            node0 = wid * BN
            lanes = lax.iota(jnp.int32, K)

            def body(g, carry):
                nbr = idx_v[pl.ds(g * K, K)]
                n = jnp.broadcast_to(node0 + g, (K,)).astype(jnp.int32)
                dx = plsc.load_gather(xv, [nbr]) - plsc.load_gather(xv, [n])
                dy = plsc.load_gather(yv, [nbr]) - plsc.load_gather(yv, [n])
                dz = plsc.load_gather(zv, [nbr]) - plsc.load_gather(zv, [n])
                s2 = dx * dx + dy * dy + dz * dz + EPS2
                y = plsc.bitcast(
                    jnp.int32(0x5F3759DF)
                    - lax.shift_right_arithmetic(plsc.bitcast(s2, jnp.int32),
                                                 1), jnp.float32)
                hx = 0.5 * s2
                for _ in range(4):
                    y = y * (1.5 - hx * y * y)
                pos = lanes * BN + g
                plsc.store_scatter(d_v, [pos], s2 * y)
                return carry

            lax.fori_loop(0, BN, body, 0)
            pltpu.sync_copy(d_v, out_hbm.at[wid, 0])

    return k(xyz, idx)


def _tc_body(x_ref, d_ref, cen_ref,
             wp_ref, bp_ref, w1_ref, b1_ref, w2_ref, b2_ref,
             w3_ref, b3_ref, w4_ref, b4_ref, out_ref, *, bn):
    def g_act(v):
        return jnp.log2(jnp.exp2(v) + 0.5)

    x = x_ref[...]
    u = jnp.transpose(d_ref[0])
    b3 = b3_ref[...] * LOG2E - 1.0
    w1e = jnp.concatenate(
        [w1_ref[...], jnp.zeros((NF, 1), jnp.float32)], axis=1)
    b1e = jnp.concatenate(
        [b1_ref[...] * LOG2E - 1.0,
         jnp.full((1, 1), 0.5849625007211562, jnp.float32)], axis=1)
    w2e = jnp.concatenate(
        [w2_ref[...], b2_ref[...] * LOG2E - 1.0], axis=0)
    w = u - cen_ref[...]
    rbf = jnp.exp2(C0 - w * w)
    g1 = g_act(jnp.dot(rbf, w1e,
                       preferred_element_type=jnp.float32) + b1e)
    g2 = g_act(jnp.dot(g1, w2e,
                       preferred_element_type=jnp.float32))
    s = jnp.sum(g2.reshape(K, bn, NF), axis=0)
    pre = jnp.dot(x, wp_ref[...],
                  preferred_element_type=jnp.float32) + bp_ref[...]
    msg = pre * s
    g3 = jnp.log(jnp.exp2(jnp.dot(msg, w3_ref[...],
                                  preferred_element_type=jnp.float32) + b3)
                 + 0.5)
    post = jnp.dot(g3, w4_ref[...], preferred_element_type=jnp.float32)
    out_ref[...] = x + post + b4_ref[...]


def _tc_main(x, d, cen, wp, bp, w1, b1, w2, b2, w3, b3, w4, b4):
    full = lambda i: (0, 0)
    return pl.pallas_call(
        functools.partial(_tc_body, bn=BN),
        grid=(NBLK,),
        in_specs=[
            pl.BlockSpec((BN, NF), lambda i: (i, 0)),
            pl.BlockSpec((1, 1, EB), lambda i: (i, 0, 0)),
            pl.BlockSpec((1, NF), full),
            pl.BlockSpec((NF, NF), full),
            pl.BlockSpec((1, NF), full),
            pl.BlockSpec((NF, NF), full),
            pl.BlockSpec((1, NF), full),
            pl.BlockSpec((NF, NF), full),
            pl.BlockSpec((1, NF), full),
            pl.BlockSpec((NF, NF), full),
            pl.BlockSpec((1, NF), full),
            pl.BlockSpec((NF, NF), full),
            pl.BlockSpec((1, NF), full),
        ],
        out_specs=pl.BlockSpec((BN, NF), lambda i: (i, 0)),
        out_shape=jax.ShapeDtypeStruct((N, NF), jnp.float32),
        compiler_params=pltpu.CompilerParams(
            dimension_semantics=("arbitrary",)),
    )(x, d, cen, wp, bp, w1, b1, w2, b2, w3, b3, w4, b4)


def kernel(x, xyz, nbr_idx, W_pre, b_pre, W1, b1, W2, b2, W3, b3, W4, b4):
    idx = nbr_idx.astype(jnp.int32).reshape(-1)
    xcols = xyz.astype(jnp.float32).T * SQG
    d = _sc_dist(xcols, idx)
    cen = (jnp.linspace(0.1, 30.1, NF).astype(jnp.float32)
           * SQG).reshape(1, NF)
    return _tc_main(x, d, cen,
                    W_pre, b_pre.reshape(1, NF),
                    W1, b1.reshape(1, NF), W2, b2.reshape(1, NF),
                    W3, b3.reshape(1, NF), W4, b4.reshape(1, NF))
